# baseline (speedup 1.0000x reference)
import jax
import jax.numpy as jnp
from jax import lax
from jax.experimental import pallas as pl
from jax.experimental.pallas import tpu as pltpu
from jax.experimental.pallas import tpu_sc as plsc

N = 10000
E = 320000
D_IN = 128
HID = 64
D_OUT = 64
HALF = 32
ALPHA = 0.1
K = 10

NC = 2
NS = 16
CHUNK = 128
NCHUNK = 160
NBUF = 4
EPT = NCHUNK * CHUNK
NP = 10240
ROWS_PT = NP // NS

_BN = 2000


def _mlp_body(x_ref, w1_ref, b1_ref, w2_ref, b2_ref, o_ref, oa_ref):
    h = jnp.dot(x_ref[...], w1_ref[...], preferred_element_type=jnp.float32)
    h = jnp.maximum(h + b1_ref[...], 0.0)
    o = jnp.dot(h, w2_ref[...], preferred_element_type=jnp.float32)
    o = o + b2_ref[...]
    o_ref[...] = o
    oa_ref[...] = o * ALPHA


def _mlp(x, W1, b1, W2, b2):
    return pl.pallas_call(
        _mlp_body,
        grid=(N // _BN,),
        in_specs=[
            pl.BlockSpec((_BN, D_IN), lambda i: (i, 0)),
            pl.BlockSpec((D_IN, HID), lambda i: (0, 0)),
            pl.BlockSpec((1, HID), lambda i: (0, 0)),
            pl.BlockSpec((HID, D_OUT), lambda i: (0, 0)),
            pl.BlockSpec((1, D_OUT), lambda i: (0, 0)),
        ],
        out_specs=[pl.BlockSpec((_BN, D_OUT), lambda i: (i, 0)),
                   pl.BlockSpec((_BN, D_OUT), lambda i: (i, 0))],
        out_shape=[jax.ShapeDtypeStruct((N, D_OUT), jnp.float32),
                   jax.ShapeDtypeStruct((N, D_OUT), jnp.float32)],
    )(x, W1, b1.reshape(1, HID), W2, b2.reshape(1, D_OUT))


def _propagate_body(h_hbm, h01_hbm, src_hbm, dst_hbm, val_hbm, out_hbm,
                    src_v, dst_v, val_v, gbufs, msgs, zacc, zbf,
                    gsems, ssems):
    c = lax.axis_index("c")
    s = lax.axis_index("s")

    pltpu.sync_copy(src_hbm.at[s], src_v)
    pltpu.sync_copy(dst_hbm.at[s], dst_v)
    pltpu.sync_copy(val_hbm.at[s], val_v)

    @pl.loop(0, NCHUNK)
    def _(j):
        @pl.loop(0, CHUNK // 16)
        def _(g):
            sl = (j, pl.ds(g * 16, 16))
            val_v[sl] = val_v[sl] * (1.0 - ALPHA)

    row0 = s * ROWS_PT

    def convert_rows(src_sp):
        @pl.loop(0, ROWS_PT // CHUNK)
        def _(p):
            base = row0 + p * CHUNK
            pltpu.sync_copy(src_sp.at[pl.ds(base, CHUNK)], msgs.at[0])

            @pl.loop(0, CHUNK, unroll=4)
            def _(r):
                a = msgs[0, r, pl.ds(0, 16)]
                b = msgs[0, r, pl.ds(16, 16)]
                gbufs[0, r, :] = plsc.pack(
                    a, b, format=plsc.PackFormat.INTERLEAVED)

            pltpu.sync_copy(gbufs.at[0], zbf.at[pl.ds(base, CHUNK)])

    pltpu.sync_copy(h_hbm.at[c, pl.ds(row0, ROWS_PT)],
                    zacc.at[pl.ds(row0, ROWS_PT)])
    convert_rows(zacc)
    pltpu.sync_copy(h01_hbm.at[c, pl.ds(row0, ROWS_PT)],
                    zacc.at[pl.ds(row0, ROWS_PT)])
    plsc.subcore_barrier()

    def scale_chunk(j, gb, mg):
        @pl.loop(0, CHUNK // 16)
        def _(g):
            vv = val_v[j, pl.ds(g * 16, 16)]
            for l in range(16):
                e = g * 16 + l
                vs = jnp.full((16,), vv[l], dtype=jnp.float32)
                row = gb[e, :]
                a, b = plsc.unpack(row, format=plsc.PackFormat.INTERLEAVED)
                mg[e, pl.ds(0, 16)] = a * vs
                mg[e, pl.ds(16, 16)] = b * vs

    @pl.loop(0, K)
    def _(t):
        for b in range(NBUF):
            pltpu.async_copy(zbf.at[src_v.at[b]], gbufs.at[b], gsems.at[b])

        @pl.loop(0, NCHUNK // NBUF)
        def _(i):
            for b in range(NBUF):
                j = i * NBUF + b
                pltpu.make_async_copy(
                    zbf.at[src_v.at[j]], gbufs.at[b], gsems.at[b]).wait()
                scale_chunk(j, gbufs.at[b], msgs.at[b])
                pltpu.async_copy(
                    msgs.at[b], zacc.at[dst_v.at[j]], ssems.at[b], add=True)

                pb = (b - 1) % NBUF
                pj = j + NBUF - 1

                @pl.when(jnp.logical_and(j >= 1, pj < NCHUNK))
                def _():
                    pltpu.make_async_copy(
                        msgs.at[pb], zacc.at[dst_v.at[j - 1]],
                        ssems.at[pb]).wait()
                    pltpu.async_copy(
                        zbf.at[src_v.at[pj]], gbufs.at[pb], gsems.at[pb])

        for b in range(NBUF):
            jd = NCHUNK - NBUF + b
            pltpu.make_async_copy(
                msgs.at[b], zacc.at[dst_v.at[jd]], ssems.at[b]).wait()

        plsc.subcore_barrier()

        @pl.when(t < K - 1)
        def _():
            convert_rows(zacc)
            pltpu.sync_copy(h01_hbm.at[c, pl.ds(row0, ROWS_PT)],
                            zacc.at[pl.ds(row0, ROWS_PT)])

        plsc.subcore_barrier()

    pltpu.sync_copy(zacc.at[pl.ds(row0, ROWS_PT)],
                    out_hbm.at[c, pl.ds(row0, ROWS_PT)])


def _propagate(h2, h012, src3, dst3, val3):
    mesh = plsc.VectorSubcoreMesh(core_axis_name="c", subcore_axis_name="s")
    cp = pltpu.CompilerParams(
        needs_layout_passes=False,
        use_tc_tiling_on_sc=False,
    )
    kfn = pl.kernel(
        _propagate_body,
        out_type=jax.ShapeDtypeStruct((NC, NP, HALF), jnp.float32),
        mesh=mesh,
        scratch_types=[
            pltpu.VMEM((NCHUNK, CHUNK), jnp.int32),
            pltpu.VMEM((NCHUNK, CHUNK), jnp.int32),
            pltpu.VMEM((NCHUNK, CHUNK), jnp.float32),
            pltpu.VMEM((NBUF, CHUNK, HALF), jnp.bfloat16),
            pltpu.VMEM((NBUF, CHUNK, HALF), jnp.float32),
            pltpu.VMEM_SHARED((NP, HALF), jnp.float32),
            pltpu.VMEM_SHARED((NP, HALF), jnp.bfloat16),
            pltpu.SemaphoreType.DMA((NBUF,)),
            pltpu.SemaphoreType.DMA((NBUF,)),
        ],
        compiler_params=cp,
    )
    return kfn(h2, h012, src3, dst3, val3)


def kernel(x, adj_indices, adj_values, W1, b1, W2, b2):
    h, h01 = _mlp(x, W1, b1, W2, b2)
    h2 = h.reshape(N, NC, HALF).transpose(1, 0, 2)
    h2 = jnp.pad(h2, ((0, 0), (0, NP - N), (0, 0)))
    h012 = h01.reshape(N, NC, HALF).transpose(1, 0, 2)
    h012 = jnp.pad(h012, ((0, 0), (0, NP - N), (0, 0)))

    pad = NS * EPT - E
    src = jnp.concatenate([adj_indices[0], jnp.zeros((pad,), jnp.int32)])
    dst = jnp.concatenate([adj_indices[1], jnp.zeros((pad,), jnp.int32)])
    val = jnp.concatenate([adj_values, jnp.zeros((pad,), jnp.float32)])
    src3 = src.reshape(NS, NCHUNK, CHUNK)
    dst3 = dst.reshape(NS, NCHUNK, CHUNK)
    val3 = val.reshape(NS, NCHUNK, CHUNK)

    z2 = _propagate(h2, h012, src3, dst3, val3)
    return z2[:, :N, :].transpose(1, 0, 2).reshape(N, D_OUT)


# speedup vs baseline: 1.6893x; 1.6893x over previous
import jax
import jax.numpy as jnp
from jax import lax
from jax.experimental import pallas as pl
from jax.experimental.pallas import tpu as pltpu
from jax.experimental.pallas import tpu_sc as plsc

N = 10000
E = 320000
D_IN = 128
HID = 64
D_OUT = 64
HALF = 32
ALPHA = 0.1
K = 10

NC = 2
NS = 16
CHUNK = 128
NCHUNK = 160
NBUF = 5
EPT = NCHUNK * CHUNK
NP = 10240
ROWS_PT = NP // NS

_BN = 2000


def _mlp_body(x_ref, w1_ref, b1_ref, w2_ref, b2_ref, o_ref, oa_ref):
    h = jnp.dot(x_ref[...], w1_ref[...], preferred_element_type=jnp.float32)
    h = jnp.maximum(h + b1_ref[...], 0.0)
    o = jnp.dot(h, w2_ref[...], preferred_element_type=jnp.float32)
    o = o + b2_ref[...]
    o_ref[...] = o
    oa_ref[...] = o * ALPHA


def _mlp(x, W1, b1, W2, b2):
    return pl.pallas_call(
        _mlp_body,
        grid=(N // _BN,),
        in_specs=[
            pl.BlockSpec((_BN, D_IN), lambda i: (i, 0)),
            pl.BlockSpec((D_IN, HID), lambda i: (0, 0)),
            pl.BlockSpec((1, HID), lambda i: (0, 0)),
            pl.BlockSpec((HID, D_OUT), lambda i: (0, 0)),
            pl.BlockSpec((1, D_OUT), lambda i: (0, 0)),
        ],
        out_specs=[pl.BlockSpec((_BN, D_OUT), lambda i: (i, 0)),
                   pl.BlockSpec((_BN, D_OUT), lambda i: (i, 0))],
        out_shape=[jax.ShapeDtypeStruct((N, D_OUT), jnp.float32),
                   jax.ShapeDtypeStruct((N, D_OUT), jnp.float32)],
    )(x, W1, b1.reshape(1, HID), W2, b2.reshape(1, D_OUT))


def _propagate_body(h_hbm, h01_hbm, src_hbm, dst_hbm, val_hbm, out_hbm,
                    src_v, dst_v, val_v, bufs, z0sp, z1sp,
                    gsems, ssems):
    c = lax.axis_index("c")
    s = lax.axis_index("s")

    pltpu.sync_copy(src_hbm.at[s], src_v)
    pltpu.sync_copy(dst_hbm.at[s], dst_v)
    pltpu.sync_copy(val_hbm.at[s], val_v)

    @pl.loop(0, NCHUNK)
    def _(j):
        @pl.loop(0, CHUNK // 16)
        def _(g):
            sl = (j, pl.ds(g * 16, 16))
            val_v[sl] = val_v[sl] * (1.0 - ALPHA)

    row0 = s * ROWS_PT
    pltpu.sync_copy(h_hbm.at[c, pl.ds(row0, ROWS_PT)],
                    z0sp.at[pl.ds(row0, ROWS_PT)])

    plsc.subcore_barrier()

    def scale_chunk(j, buf):
        @pl.loop(0, CHUNK // 16)
        def _(g):
            vv = val_v[j, pl.ds(g * 16, 16)]
            for l in range(16):
                e = g * 16 + l
                vs = jnp.full((16,), vv[l], dtype=jnp.float32)
                a = buf[e, pl.ds(0, 16)]
                buf[e, pl.ds(0, 16)] = a * vs
                b = buf[e, pl.ds(16, 16)]
                buf[e, pl.ds(16, 16)] = b * vs

    def one_step(cur, nxt):
        pltpu.sync_copy(h01_hbm.at[c, pl.ds(row0, ROWS_PT)],
                        nxt.at[pl.ds(row0, ROWS_PT)])
        plsc.subcore_barrier()

        for b in range(NBUF):
            pltpu.async_copy(cur.at[src_v.at[b]], bufs.at[b], gsems.at[b])

        @pl.loop(0, NCHUNK // NBUF)
        def _(i):
            for b in range(NBUF):
                j = i * NBUF + b
                pltpu.make_async_copy(
                    cur.at[src_v.at[j]], bufs.at[b], gsems.at[b]).wait()
                scale_chunk(j, bufs.at[b])
                pltpu.async_copy(
                    bufs.at[b], nxt.at[dst_v.at[j]], ssems.at[b], add=True)

                pb = (b - 1) % NBUF
                pj = j + NBUF - 1

                @pl.when(jnp.logical_and(j >= 1, pj < NCHUNK))
                def _():
                    pltpu.make_async_copy(
                        bufs.at[pb], nxt.at[dst_v.at[j - 1]],
                        ssems.at[pb]).wait()
                    pltpu.async_copy(
                        cur.at[src_v.at[pj]], bufs.at[pb], gsems.at[pb])

        for b in range(NBUF):
            jd = NCHUNK - NBUF + b
            pltpu.make_async_copy(
                bufs.at[b], nxt.at[dst_v.at[jd]], ssems.at[b]).wait()

        plsc.subcore_barrier()

    @pl.loop(0, K // 2)
    def _(t2):
        one_step(z0sp, z1sp)
        one_step(z1sp, z0sp)

    pltpu.sync_copy(z0sp.at[pl.ds(row0, ROWS_PT)],
                    out_hbm.at[c, pl.ds(row0, ROWS_PT)])


def _propagate(h2, h012, src3, dst3, val3):
    mesh = plsc.VectorSubcoreMesh(core_axis_name="c", subcore_axis_name="s")
    cp = pltpu.CompilerParams(
        needs_layout_passes=False,
        use_tc_tiling_on_sc=False,
    )
    kfn = pl.kernel(
        _propagate_body,
        out_type=jax.ShapeDtypeStruct((NC, NP, HALF), jnp.float32),
        mesh=mesh,
        scratch_types=[
            pltpu.VMEM((NCHUNK, CHUNK), jnp.int32),
            pltpu.VMEM((NCHUNK, CHUNK), jnp.int32),
            pltpu.VMEM((NCHUNK, CHUNK), jnp.float32),
            pltpu.VMEM((NBUF, CHUNK, HALF), jnp.float32),
            pltpu.VMEM_SHARED((NP, HALF), jnp.float32),
            pltpu.VMEM_SHARED((NP, HALF), jnp.float32),
            pltpu.SemaphoreType.DMA((NBUF,)),
            pltpu.SemaphoreType.DMA((NBUF,)),
        ],
        compiler_params=cp,
    )
    return kfn(h2, h012, src3, dst3, val3)


def kernel(x, adj_indices, adj_values, W1, b1, W2, b2):
    h, h01 = _mlp(x, W1, b1, W2, b2)
    h2 = h.reshape(N, NC, HALF).transpose(1, 0, 2)
    h2 = jnp.pad(h2, ((0, 0), (0, NP - N), (0, 0)))
    h012 = h01.reshape(N, NC, HALF).transpose(1, 0, 2)
    h012 = jnp.pad(h012, ((0, 0), (0, NP - N), (0, 0)))

    pad = NS * EPT - E
    src = jnp.concatenate([adj_indices[0], jnp.zeros((pad,), jnp.int32)])
    dst = jnp.concatenate([adj_indices[1], jnp.zeros((pad,), jnp.int32)])
    val = jnp.concatenate([adj_values, jnp.zeros((pad,), jnp.float32)])
    src3 = src.reshape(NS, NCHUNK, CHUNK)
    dst3 = dst.reshape(NS, NCHUNK, CHUNK)
    val3 = val.reshape(NS, NCHUNK, CHUNK)

    z2 = _propagate(h2, h012, src3, dst3, val3)
    return z2[:, :N, :].transpose(1, 0, 2).reshape(N, D_OUT)


# speedup vs baseline: 1.6904x; 1.0006x over previous
import jax
import jax.numpy as jnp
from jax import lax
from jax.experimental import pallas as pl
from jax.experimental.pallas import tpu as pltpu
from jax.experimental.pallas import tpu_sc as plsc

N = 10000
E = 320000
D_IN = 128
HID = 64
D_OUT = 64
HALF = 32
ALPHA = 0.1
K = 10

NC = 2
NS = 16
CHUNK = 128
NCHUNK = 160
NBUF = 4
EPT = NCHUNK * CHUNK
NP = 10240
ROWS_PT = NP // NS

_BN = 2000


def _mlp_body(x_ref, w1_ref, b1_ref, w2_ref, b2_ref, o_ref, oa_ref):
    h = jnp.dot(x_ref[...], w1_ref[...], preferred_element_type=jnp.float32)
    h = jnp.maximum(h + b1_ref[...], 0.0)
    o = jnp.dot(h, w2_ref[...], preferred_element_type=jnp.float32)
    o = o + b2_ref[...]
    o_ref[...] = o
    oa_ref[...] = o * ALPHA


def _mlp(x, W1, b1, W2, b2):
    return pl.pallas_call(
        _mlp_body,
        grid=(N // _BN,),
        in_specs=[
            pl.BlockSpec((_BN, D_IN), lambda i: (i, 0)),
            pl.BlockSpec((D_IN, HID), lambda i: (0, 0)),
            pl.BlockSpec((1, HID), lambda i: (0, 0)),
            pl.BlockSpec((HID, D_OUT), lambda i: (0, 0)),
            pl.BlockSpec((1, D_OUT), lambda i: (0, 0)),
        ],
        out_specs=[pl.BlockSpec((_BN, D_OUT), lambda i: (i, 0)),
                   pl.BlockSpec((_BN, D_OUT), lambda i: (i, 0))],
        out_shape=[jax.ShapeDtypeStruct((N, D_OUT), jnp.float32),
                   jax.ShapeDtypeStruct((N, D_OUT), jnp.float32)],
    )(x, W1, b1.reshape(1, HID), W2, b2.reshape(1, D_OUT))


def _propagate_body(h_hbm, h01_hbm, src_hbm, dst_hbm, val_hbm, out_hbm,
                    src_v, dst_v, val_v, bufs, z0sp, z1sp,
                    gsems, ssems):
    c = lax.axis_index("c")
    s = lax.axis_index("s")

    pltpu.sync_copy(src_hbm.at[s], src_v)
    pltpu.sync_copy(dst_hbm.at[s], dst_v)
    pltpu.sync_copy(val_hbm.at[s], val_v)

    @pl.loop(0, NCHUNK)
    def _(j):
        @pl.loop(0, CHUNK // 16)
        def _(g):
            sl = (j, pl.ds(g * 16, 16))
            val_v[sl] = val_v[sl] * (1.0 - ALPHA)

    row0 = s * ROWS_PT
    pltpu.sync_copy(h_hbm.at[c, pl.ds(row0, ROWS_PT)],
                    z0sp.at[pl.ds(row0, ROWS_PT)])

    plsc.subcore_barrier()

    def scale_chunk(j, buf):
        @pl.loop(0, CHUNK // 16, unroll=2)
        def _(g):
            vv = val_v[j, pl.ds(g * 16, 16)]
            for l in range(16):
                e = g * 16 + l
                vs = jnp.full((16,), vv[l], dtype=jnp.float32)
                a = buf[e, pl.ds(0, 16)]
                buf[e, pl.ds(0, 16)] = a * vs
                b = buf[e, pl.ds(16, 16)]
                buf[e, pl.ds(16, 16)] = b * vs

    def one_step(cur, nxt):
        pltpu.sync_copy(h01_hbm.at[c, pl.ds(row0, ROWS_PT)],
                        nxt.at[pl.ds(row0, ROWS_PT)])
        plsc.subcore_barrier()

        for b in range(NBUF):
            pltpu.async_copy(cur.at[src_v.at[b]], bufs.at[b], gsems.at[b])

        @pl.loop(0, NCHUNK // NBUF)
        def _(i):
            for b in range(NBUF):
                j = i * NBUF + b
                pltpu.make_async_copy(
                    cur.at[src_v.at[j]], bufs.at[b], gsems.at[b]).wait()
                scale_chunk(j, bufs.at[b])
                pltpu.async_copy(
                    bufs.at[b], nxt.at[dst_v.at[j]], ssems.at[b], add=True)

                pb = (b - 1) % NBUF
                pj = j + NBUF - 1

                @pl.when(jnp.logical_and(j >= 1, pj < NCHUNK))
                def _():
                    pltpu.make_async_copy(
                        bufs.at[pb], nxt.at[dst_v.at[j - 1]],
                        ssems.at[pb]).wait()
                    pltpu.async_copy(
                        cur.at[src_v.at[pj]], bufs.at[pb], gsems.at[pb])

        for b in range(NBUF):
            jd = NCHUNK - NBUF + b
            pltpu.make_async_copy(
                bufs.at[b], nxt.at[dst_v.at[jd]], ssems.at[b]).wait()

        plsc.subcore_barrier()

    @pl.loop(0, K // 2)
    def _(t2):
        one_step(z0sp, z1sp)
        one_step(z1sp, z0sp)

    pltpu.sync_copy(z0sp.at[pl.ds(row0, ROWS_PT)],
                    out_hbm.at[c, pl.ds(row0, ROWS_PT)])


def _propagate(h2, h012, src3, dst3, val3):
    mesh = plsc.VectorSubcoreMesh(core_axis_name="c", subcore_axis_name="s")
    cp = pltpu.CompilerParams(
        needs_layout_passes=False,
        use_tc_tiling_on_sc=False,
    )
    kfn = pl.kernel(
        _propagate_body,
        out_type=jax.ShapeDtypeStruct((NC, NP, HALF), jnp.float32),
        mesh=mesh,
        scratch_types=[
            pltpu.VMEM((NCHUNK, CHUNK), jnp.int32),
            pltpu.VMEM((NCHUNK, CHUNK), jnp.int32),
            pltpu.VMEM((NCHUNK, CHUNK), jnp.float32),
            pltpu.VMEM((NBUF, CHUNK, HALF), jnp.float32),
            pltpu.VMEM_SHARED((NP, HALF), jnp.float32),
            pltpu.VMEM_SHARED((NP, HALF), jnp.float32),
            pltpu.SemaphoreType.DMA((NBUF,)),
            pltpu.SemaphoreType.DMA((NBUF,)),
        ],
        compiler_params=cp,
    )
    return kfn(h2, h012, src3, dst3, val3)


def kernel(x, adj_indices, adj_values, W1, b1, W2, b2):
    h, h01 = _mlp(x, W1, b1, W2, b2)
    h2 = h.reshape(N, NC, HALF).transpose(1, 0, 2)
    h2 = jnp.pad(h2, ((0, 0), (0, NP - N), (0, 0)))
    h012 = h01.reshape(N, NC, HALF).transpose(1, 0, 2)
    h012 = jnp.pad(h012, ((0, 0), (0, NP - N), (0, 0)))

    pad = NS * EPT - E
    src = jnp.concatenate([adj_indices[0], jnp.zeros((pad,), jnp.int32)])
    dst = jnp.concatenate([adj_indices[1], jnp.zeros((pad,), jnp.int32)])
    val = jnp.concatenate([adj_values, jnp.zeros((pad,), jnp.float32)])
    src3 = src.reshape(NS, NCHUNK, CHUNK)
    dst3 = dst.reshape(NS, NCHUNK, CHUNK)
    val3 = val.reshape(NS, NCHUNK, CHUNK)

    z2 = _propagate(h2, h012, src3, dst3, val3)
    return z2[:, :N, :].transpose(1, 0, 2).reshape(N, D_OUT)


# speedup vs baseline: 1.6996x; 1.0055x over previous
import jax
import jax.numpy as jnp
from jax import lax
from jax.experimental import pallas as pl
from jax.experimental.pallas import tpu as pltpu
from jax.experimental.pallas import tpu_sc as plsc

N = 10000
E = 320000
D_IN = 128
HID = 64
D_OUT = 64
HALF = 32
ALPHA = 0.1
K = 10

NC = 2
NS = 16
CHUNK = 128
NCHUNK = 160
NBUF = 4
EPT = NCHUNK * CHUNK
NP = 10240
ROWS_PT = NP // NS

_BN = 2000


def _mlp_body(x_ref, w1_ref, b1_ref, w2_ref, b2_ref, o_ref, oa_ref):
    h = jnp.dot(x_ref[...], w1_ref[...], preferred_element_type=jnp.float32)
    h = jnp.maximum(h + b1_ref[...], 0.0)
    o = jnp.dot(h, w2_ref[...], preferred_element_type=jnp.float32)
    o = o + b2_ref[...]
    o_ref[...] = o
    oa_ref[...] = o * ALPHA


def _mlp(x, W1, b1, W2, b2):
    return pl.pallas_call(
        _mlp_body,
        grid=(N // _BN,),
        in_specs=[
            pl.BlockSpec((_BN, D_IN), lambda i: (i, 0)),
            pl.BlockSpec((D_IN, HID), lambda i: (0, 0)),
            pl.BlockSpec((1, HID), lambda i: (0, 0)),
            pl.BlockSpec((HID, D_OUT), lambda i: (0, 0)),
            pl.BlockSpec((1, D_OUT), lambda i: (0, 0)),
        ],
        out_specs=[pl.BlockSpec((_BN, D_OUT), lambda i: (i, 0)),
                   pl.BlockSpec((_BN, D_OUT), lambda i: (i, 0))],
        out_shape=[jax.ShapeDtypeStruct((N, D_OUT), jnp.float32),
                   jax.ShapeDtypeStruct((N, D_OUT), jnp.float32)],
    )(x, W1, b1.reshape(1, HID), W2, b2.reshape(1, D_OUT))


def _propagate_body(h_hbm, h01_hbm, src_hbm, dst_hbm, val_hbm, out_hbm,
                    src_v, dst_v, val_v, bufs, z0sp, z1sp,
                    gsems, ssems):
    c = lax.axis_index("c")
    s = lax.axis_index("s")

    pltpu.sync_copy(src_hbm.at[s], src_v)
    pltpu.sync_copy(dst_hbm.at[s], dst_v)
    pltpu.sync_copy(val_hbm.at[s], val_v)

    @pl.loop(0, NCHUNK)
    def _(j):
        @pl.loop(0, CHUNK // 16)
        def _(g):
            sl = (j, pl.ds(g * 16, 16))
            val_v[sl] = val_v[sl] * (1.0 - ALPHA)

    row0 = s * ROWS_PT
    pltpu.sync_copy(h_hbm.at[c, pl.ds(row0, ROWS_PT)],
                    z0sp.at[pl.ds(row0, ROWS_PT)])

    plsc.subcore_barrier()

    def scale_chunk(j, buf):
        @pl.loop(0, CHUNK // 16)
        def _(g):
            vv = val_v[j, pl.ds(g * 16, 16)]
            for l in range(16):
                e = g * 16 + l
                vs = jnp.full((16,), vv[l], dtype=jnp.float32)
                a = buf[e, pl.ds(0, 16)]
                buf[e, pl.ds(0, 16)] = a * vs
                b = buf[e, pl.ds(16, 16)]
                buf[e, pl.ds(16, 16)] = b * vs

    def one_step(cur, nxt):
        pltpu.sync_copy(h01_hbm.at[c, pl.ds(row0, ROWS_PT)],
                        nxt.at[pl.ds(row0, ROWS_PT)])
        plsc.subcore_barrier()

        for b in range(NBUF):
            pltpu.async_copy(cur.at[src_v.at[b]], bufs.at[b], gsems.at[b])

        @pl.loop(0, NCHUNK // NBUF)
        def _(i):
            for b in range(NBUF):
                j = i * NBUF + b
                pltpu.make_async_copy(
                    cur.at[src_v.at[j]], bufs.at[b], gsems.at[b]).wait()
                scale_chunk(j, bufs.at[b])
                pltpu.async_copy(
                    bufs.at[b], nxt.at[dst_v.at[j]], ssems.at[b], add=True)

                pb = (b - 1) % NBUF
                pj = j + NBUF - 1

                @pl.when(jnp.logical_and(j >= 1, pj < NCHUNK))
                def _():
                    pltpu.make_async_copy(
                        bufs.at[pb], nxt.at[dst_v.at[j - 1]],
                        ssems.at[pb]).wait()
                    pltpu.async_copy(
                        cur.at[src_v.at[pj]], bufs.at[pb], gsems.at[pb])

        for b in range(NBUF):
            jd = NCHUNK - NBUF + b
            pltpu.make_async_copy(
                bufs.at[b], nxt.at[dst_v.at[jd]], ssems.at[b]).wait()

        plsc.subcore_barrier()

    @pl.loop(0, K // 2)
    def _(t2):
        one_step(z0sp, z1sp)
        one_step(z1sp, z0sp)

    pltpu.sync_copy(z0sp.at[pl.ds(row0, ROWS_PT)],
                    out_hbm.at[c, pl.ds(row0, ROWS_PT)])


def _propagate(h2, h012, src3, dst3, val3):
    mesh = plsc.VectorSubcoreMesh(core_axis_name="c", subcore_axis_name="s")
    cp = pltpu.CompilerParams(
        needs_layout_passes=False,
        use_tc_tiling_on_sc=False,
    )
    kfn = pl.kernel(
        _propagate_body,
        out_type=jax.ShapeDtypeStruct((NC, NP, HALF), jnp.float32),
        mesh=mesh,
        scratch_types=[
            pltpu.VMEM((NCHUNK, CHUNK), jnp.int32),
            pltpu.VMEM((NCHUNK, CHUNK), jnp.int32),
            pltpu.VMEM((NCHUNK, CHUNK), jnp.float32),
            pltpu.VMEM((NBUF, CHUNK, HALF), jnp.float32),
            pltpu.VMEM_SHARED((NP, HALF), jnp.float32),
            pltpu.VMEM_SHARED((NP, HALF), jnp.float32),
            pltpu.SemaphoreType.DMA((NBUF,)),
            pltpu.SemaphoreType.DMA((NBUF,)),
        ],
        compiler_params=cp,
    )
    return kfn(h2, h012, src3, dst3, val3)


def kernel(x, adj_indices, adj_values, W1, b1, W2, b2):
    h, h01 = _mlp(x, W1, b1, W2, b2)
    h2 = h.reshape(N, NC, HALF).transpose(1, 0, 2)
    h2 = jnp.pad(h2, ((0, 0), (0, NP - N), (0, 0)))
    h012 = h01.reshape(N, NC, HALF).transpose(1, 0, 2)
    h012 = jnp.pad(h012, ((0, 0), (0, NP - N), (0, 0)))

    pad = NS * EPT - E
    src = jnp.concatenate([adj_indices[0], jnp.zeros((pad,), jnp.int32)])
    dst = jnp.concatenate([adj_indices[1], jnp.zeros((pad,), jnp.int32)])
    val = jnp.concatenate([adj_values, jnp.zeros((pad,), jnp.float32)])
    src3 = src.reshape(NS, NCHUNK, CHUNK)
    dst3 = dst.reshape(NS, NCHUNK, CHUNK)
    val3 = val.reshape(NS, NCHUNK, CHUNK)

    z2 = _propagate(h2, h012, src3, dst3, val3)
    return z2[:, :N, :].transpose(1, 0, 2).reshape(N, D_OUT)
